# baseline (speedup 1.0000x reference)
import jax
import jax.numpy as jnp
from jax.experimental import pallas as pl
from jax.experimental.pallas import tpu as pltpu

_B, _N, _C = 64, 8732, 21
_G = 4
_BC = _B // _G
_NPAD = 8832
_ONE_BITS_P1 = 0x3F800001


def _chunk_kernel(yt_ref, loc_ref, log_ref, hard_ref, closs_ref, stats_ref):
    yt = yt_ref[0]
    lt = yt[0:4]
    ct = yt[4:25]
    m = yt[25:26]
    x = log_ref[0]

    mx = jnp.max(x, axis=0, keepdims=True)
    ex = jnp.exp(x - mx)
    se = jnp.sum(ex, axis=0, keepdims=True)
    logp = (x - mx) - jnp.log(se)
    closs = -jnp.sum(jnp.maximum(ct, 1e-7) * logp, axis=0, keepdims=True)

    yp = loc_ref[0]
    diff = jnp.abs(lt - yp)
    l1 = jnp.where(diff < 1.0, 0.5 * diff * diff, diff - 0.5)
    lloss = jnp.sum(l1, axis=0, keepdims=True)

    np_s = jnp.sum(m)
    pc_s = jnp.sum(closs * m)
    pll_s = jnp.sum(lloss * m)
    hard = ((se - ex[0:1]) / se) * (1.0 - m)

    hard_ref[0, 0:1, pl.ds(0, _N)] = hard
    hard_ref[0, 0:1, pl.ds(_N, _NPAD - _N)] = jnp.full(
        (1, _NPAD - _N), -1.0, jnp.float32)
    closs_ref[0, 0:1, pl.ds(0, _N)] = closs
    closs_ref[0, 0:1, pl.ds(_N, _NPAD - _N)] = jnp.zeros(
        (1, _NPAD - _N), jnp.float32)

    lane = jax.lax.broadcasted_iota(jnp.int32, (1, 128), 1)
    stats_ref[0, 0:1, :] = jnp.where(
        lane == 0, np_s,
        jnp.where(lane == 1, pc_s, jnp.where(lane == 2, pll_s, 0.0)))


def _select_kernel(*refs):
    h_refs = refs[0:_G]
    c_refs = refs[_G:2 * _G]
    s_refs = refs[2 * _G:3 * _G]
    out_ref = refs[3 * _G]

    bits = [jax.lax.bitcast_convert_type(r[...], jnp.int32) for r in h_refs]
    cls = [r[...] for r in c_refs]
    npv = jnp.concatenate([r[:, 0:1] for r in s_refs], axis=0)

    nn = jnp.minimum(3.0 * npv, float(_N) - npv)
    has = jnp.sum((nn > 0.0).astype(jnp.float32))
    kf = jnp.where(has > 0.0, jnp.sum(nn), 100.0)
    k = kf.astype(jnp.int32)

    def count_ge(th):
        c = jnp.int32(0)
        for bg in bits:
            c = c + jnp.sum((bg >= th).astype(jnp.int32))
        return c

    def vbody(_, carry):
        lo, hi = carry
        mid = lo + (hi - lo) // 2
        keep = count_ge(mid) >= k
        return (jnp.where(keep, mid, lo), jnp.where(keep, hi, mid))

    lo, _hi = jax.lax.fori_loop(
        0, 31, vbody, (jnp.int32(-1), jnp.int32(_ONE_BITS_P1)))
    t = lo
    cnt_ge = count_ge(t)

    def _no_ties(_):
        s = jnp.float32(0.0)
        for bg, cg in zip(bits, cls):
            s = s + jnp.sum(jnp.where(bg >= t, cg, 0.0))
        return s

    def _with_ties(_):
        fis = [(jax.lax.broadcasted_iota(jnp.int32, (_BC, _NPAD), 0)
                + g * _BC) * _N
               + jax.lax.broadcasted_iota(jnp.int32, (_BC, _NPAD), 1)
               for g in range(_G)]
        cnt_gt = jnp.int32(0)
        for bg in bits:
            cnt_gt = cnt_gt + jnp.sum((bg > t).astype(jnp.int32))
        r = k - cnt_gt

        def ibody(_, carry):
            lo_i, hi_i = carry
            mid = lo_i + (hi_i - lo_i) // 2
            c = jnp.int32(0)
            for bg, fg in zip(bits, fis):
                c = c + jnp.sum(((bg == t) & (fg < mid)).astype(jnp.int32))
            ge = c >= r
            return (jnp.where(ge, lo_i, mid), jnp.where(ge, mid, hi_i))

        _lo_i, m_i = jax.lax.fori_loop(
            0, 20, ibody, (jnp.int32(0), jnp.int32(_B * _N)))
        s = jnp.float32(0.0)
        for bg, cg, fg in zip(bits, cls, fis):
            s = s + jnp.sum(jnp.where(bg > t, cg, 0.0))
            s = s + jnp.sum(jnp.where((bg == t) & (fg < m_i), cg, 0.0))
        return s

    neg = jax.lax.cond(cnt_ge == k, _no_ties, _with_ties, 0)
    denom = jnp.sum(jnp.where(npv != 0.0, npv, 1.0))
    pc_t = jnp.float32(0.0)
    pl_t = jnp.float32(0.0)
    for r in s_refs:
        pc_t = pc_t + jnp.sum(r[:, 1:2])
        pl_t = pl_t + jnp.sum(r[:, 2:3])
    out_ref[...] = ((pc_t + neg + pl_t) / denom).reshape(1, 1)


def _build_chunk_call(interpret=False):
    return pl.pallas_call(
        _chunk_kernel,
        grid=(_BC,),
        in_specs=[
            pl.BlockSpec((1, 26, _N), lambda b: (b, 0, 0)),
            pl.BlockSpec((1, 4, _N), lambda b: (b, 0, 0)),
            pl.BlockSpec((1, _C, _N), lambda b: (b, 0, 0)),
        ],
        out_specs=[
            pl.BlockSpec((1, 1, _NPAD), lambda b: (b, 0, 0)),
            pl.BlockSpec((1, 1, _NPAD), lambda b: (b, 0, 0)),
            pl.BlockSpec((1, 1, 128), lambda b: (b, 0, 0)),
        ],
        out_shape=[
            jax.ShapeDtypeStruct((_BC, 1, _NPAD), jnp.float32),
            jax.ShapeDtypeStruct((_BC, 1, _NPAD), jnp.float32),
            jax.ShapeDtypeStruct((_BC, 1, 128), jnp.float32),
        ],
        compiler_params=pltpu.CompilerParams(
            dimension_semantics=("arbitrary",),
            vmem_limit_bytes=100 * 1024 * 1024,
        ),
        interpret=interpret,
    )


def _build_select_call(interpret=False):
    full2 = lambda b: (0, 0)
    return pl.pallas_call(
        _select_kernel,
        grid=(1,),
        in_specs=(
            [pl.BlockSpec((_BC, _NPAD), full2)] * (2 * _G)
            + [pl.BlockSpec((_BC, 128), full2)] * _G
        ),
        out_specs=pl.BlockSpec((1, 1), full2),
        out_shape=jax.ShapeDtypeStruct((1, 1), jnp.float32),
        compiler_params=pltpu.CompilerParams(
            dimension_semantics=("arbitrary",),
            vmem_limit_bytes=100 * 1024 * 1024,
        ),
        interpret=interpret,
    )


@jax.jit
def kernel(y_true, y_pred_loc, y_pred_logits):
    hs, cs, ss = [], [], []
    chunk = _build_chunk_call()
    for g in range(_G):
        sl = slice(_BC * g, _BC * (g + 1))
        h, c, s = chunk(
            jnp.swapaxes(y_true[sl], 1, 2),
            jnp.swapaxes(y_pred_loc[sl], 1, 2),
            jnp.swapaxes(y_pred_logits[sl], 1, 2))
        hs.append(h.reshape(_BC, _NPAD))
        cs.append(c.reshape(_BC, _NPAD))
        ss.append(s.reshape(_BC, 128))
    out = _build_select_call()(*hs, *cs, *ss)
    return out[0, 0]


# speedup vs baseline: 1.3567x; 1.3567x over previous
import jax
import jax.numpy as jnp
from jax.experimental import pallas as pl
from jax.experimental.pallas import tpu as pltpu

_B, _N, _C = 64, 8732, 21
_NPAD = 8832
_ONE_BITS_P1 = 0x3F800001


def _mbl_kernel(yt_ref, loc_ref, log_ref, out_ref, hard_ref, closs_ref, stats_ref):
    b = pl.program_id(0)

    @pl.when(b == 0)
    def _init_pads():
        hard_ref[:, pl.ds(_N, _NPAD - _N)] = jnp.full(
            (_B, _NPAD - _N), -1.0, jnp.float32)
        closs_ref[:, pl.ds(_N, _NPAD - _N)] = jnp.zeros(
            (_B, _NPAD - _N), jnp.float32)

    yt = yt_ref[0]
    lt = yt[0:4]
    ct = yt[4:25]
    m = yt[25:26]
    x = log_ref[0]

    mx = jnp.max(x, axis=0, keepdims=True)
    ex = jnp.exp(x - mx)
    se = jnp.sum(ex, axis=0, keepdims=True)
    logp = (x - mx) - jnp.log(se)
    closs = -jnp.sum(jnp.maximum(ct, 1e-7) * logp, axis=0, keepdims=True)

    yp = loc_ref[0]
    diff = jnp.abs(lt - yp)
    l1 = jnp.where(diff < 1.0, 0.5 * diff * diff, diff - 0.5)
    lloss = jnp.sum(l1, axis=0, keepdims=True)

    np_s = jnp.sum(m)
    pc_s = jnp.sum(closs * m)
    pll_s = jnp.sum(lloss * m)
    hard = ((se - ex[0:1]) / se) * (1.0 - m)

    hard_ref[pl.ds(b, 1), pl.ds(0, _N)] = hard
    closs_ref[pl.ds(b, 1), pl.ds(0, _N)] = closs

    lane = jax.lax.broadcasted_iota(jnp.int32, (1, 128), 1)
    row = jnp.where(lane == 0, np_s,
                    jnp.where(lane == 1, pc_s,
                              jnp.where(lane == 2, pll_s, 0.0)))
    stats_ref[pl.ds(b, 1), :] = row

    @pl.when(b == _B - 1)
    def _selection():
        bits = jax.lax.bitcast_convert_type(hard_ref[...], jnp.int32)
        npv = stats_ref[:, 0:1]
        nn = jnp.minimum(3.0 * npv, float(_N) - npv)
        has = jnp.sum((nn > 0.0).astype(jnp.float32))
        kf = jnp.where(has > 0.0, jnp.sum(nn), 100.0)
        k = kf.astype(jnp.int32)

        def vbody(_, carry):
            lo, hi = carry
            mid = lo + (hi - lo) // 2
            c = jnp.sum((bits >= mid).astype(jnp.int32))
            keep = c >= k
            return (jnp.where(keep, mid, lo), jnp.where(keep, hi, mid))

        lo, _hi = jax.lax.fori_loop(
            0, 31, vbody, (jnp.int32(-1), jnp.int32(_ONE_BITS_P1)))
        t = lo
        cl = closs_ref[...]
        cnt_ge = jnp.sum((bits >= t).astype(jnp.int32))

        def _no_ties(_):
            return jnp.sum(jnp.where(bits >= t, cl, 0.0))

        def _with_ties(_):
            gt = bits > t
            tied = bits == t
            cnt_gt = jnp.sum(gt.astype(jnp.int32))
            r = k - cnt_gt
            fi = (jax.lax.broadcasted_iota(jnp.int32, (_B, _NPAD), 0) * _N
                  + jax.lax.broadcasted_iota(jnp.int32, (_B, _NPAD), 1))

            def ibody(_, carry):
                lo_i, hi_i = carry
                mid = lo_i + (hi_i - lo_i) // 2
                c = jnp.sum((tied & (fi < mid)).astype(jnp.int32))
                ge = c >= r
                return (jnp.where(ge, lo_i, mid), jnp.where(ge, mid, hi_i))

            _lo_i, m_i = jax.lax.fori_loop(
                0, 20, ibody, (jnp.int32(0), jnp.int32(_B * _N)))
            return (jnp.sum(jnp.where(gt, cl, 0.0))
                    + jnp.sum(jnp.where(tied & (fi < m_i), cl, 0.0)))

        neg = jax.lax.cond(cnt_ge == k, _no_ties, _with_ties, 0)
        denom = jnp.sum(jnp.where(npv != 0.0, npv, 1.0))
        pc_t = jnp.sum(stats_ref[:, 1:2])
        pl_t = jnp.sum(stats_ref[:, 2:3])
        out_ref[...] = ((pc_t + neg + pl_t) / denom).reshape(1, 1)


def _build_call(interpret=False):
    return pl.pallas_call(
        _mbl_kernel,
        grid=(_B,),
        in_specs=[
            pl.BlockSpec((1, 26, _N), lambda b: (b, 0, 0)),
            pl.BlockSpec((1, 4, _N), lambda b: (b, 0, 0)),
            pl.BlockSpec((1, _C, _N), lambda b: (b, 0, 0)),
        ],
        out_specs=pl.BlockSpec((1, 1), lambda b: (0, 0)),
        out_shape=jax.ShapeDtypeStruct((1, 1), jnp.float32),
        scratch_shapes=[
            pltpu.VMEM((_B, _NPAD), jnp.float32),
            pltpu.VMEM((_B, _NPAD), jnp.float32),
            pltpu.VMEM((_B, 128), jnp.float32),
        ],
        compiler_params=pltpu.CompilerParams(
            dimension_semantics=("arbitrary",),
            vmem_limit_bytes=100 * 1024 * 1024,
        ),
        interpret=interpret,
    )


@jax.jit
def kernel(y_true, y_pred_loc, y_pred_logits):
    ytt = jnp.swapaxes(y_true, 1, 2)
    loct = jnp.swapaxes(y_pred_loc, 1, 2)
    logt = jnp.swapaxes(y_pred_logits, 1, 2)
    out = _build_call()(ytt, loct, logt)
    return out[0, 0]
